# baseline (speedup 1.0000x reference)
import functools

import jax
import jax.numpy as jnp
from jax import lax
from jax.experimental import pallas as pl
from jax.experimental.pallas import tpu as pltpu
from jax.experimental.pallas import tpu_sc as plsc

_NW = 32
_CHUNK = 16384
_LANES = 16
_UNROLL = 8


def _add_chunk(xbuf, pbuf, obuf):
    def add_body(j, c):
        base = j * (_LANES * _UNROLL)
        for u in range(_UNROLL):
            sl = pl.ds(base + u * _LANES, _LANES)
            obuf[sl] = xbuf[sl] + pbuf[sl]
        return c

    lax.fori_loop(0, _CHUNK // (_LANES * _UNROLL), add_body, 0)


def _sc_body(x_hbm, pos_hbm, out_hbm,
             xb0, pb0, ob0, xb1, pb1, ob1,
             sin0, sin1, sout0, sout1,
             *, per_w, pos_words):
    wid = lax.axis_index("s") * 2 + lax.axis_index("c")
    xoff = pl.multiple_of(wid * per_w, _CHUNK)
    poff = pl.multiple_of((wid * per_w) % pos_words, _CHUNK)
    n_pairs = per_w // (2 * _CHUNK)

    def in_copy(k, xb, pb, sem):
        base = pl.multiple_of(xoff + k * _CHUNK, _CHUNK)
        pbase = pl.multiple_of(poff + k * _CHUNK, _CHUNK)
        cx = pltpu.make_async_copy(x_hbm.at[pl.ds(base, _CHUNK)], xb, sem)
        cp = pltpu.make_async_copy(pos_hbm.at[pl.ds(pbase, _CHUNK)], pb, sem)
        cx.start()
        cp.start()
        return cx, cp

    def out_copy(k, ob, sem):
        base = pl.multiple_of(xoff + k * _CHUNK, _CHUNK)
        c = pltpu.make_async_copy(ob, out_hbm.at[pl.ds(base, _CHUNK)], sem)
        c.start()
        return c

    cx0, cp0 = in_copy(0, xb0, pb0, sin0)
    cx1, cp1 = in_copy(1, xb1, pb1, sin1)

    def pair_body(j, c):
        k0 = j * 2
        pltpu.make_async_copy(x_hbm.at[pl.ds(0, _CHUNK)], xb0, sin0).wait()
        pltpu.make_async_copy(pos_hbm.at[pl.ds(0, _CHUNK)], pb0, sin0).wait()
        _add_chunk(xb0, pb0, ob0)
        o0 = out_copy(k0, ob0, sout0)

        pltpu.make_async_copy(x_hbm.at[pl.ds(0, _CHUNK)], xb1, sin1).wait()
        pltpu.make_async_copy(pos_hbm.at[pl.ds(0, _CHUNK)], pb1, sin1).wait()
        _add_chunk(xb1, pb1, ob1)
        o1 = out_copy(k0 + 1, ob1, sout1)

        @pl.when(j + 1 < n_pairs)
        def _():
            in_copy(k0 + 2, xb0, pb0, sin0)
            in_copy(k0 + 3, xb1, pb1, sin1)

        pltpu.make_async_copy(ob0, out_hbm.at[pl.ds(0, _CHUNK)], sout0).wait()
        pltpu.make_async_copy(ob1, out_hbm.at[pl.ds(0, _CHUNK)], sout1).wait()
        return c

    lax.fori_loop(0, n_pairs, pair_body, 0)


def kernel(x, position_embeddings):
    B, T, C = x.shape
    pos = position_embeddings[:T]
    x_words = B * T * C
    pos_words = T * C
    per_w = x_words // _NW

    mesh = plsc.VectorSubcoreMesh(core_axis_name="c", subcore_axis_name="s")
    sc_call = pl.kernel(
        functools.partial(_sc_body, per_w=per_w, pos_words=pos_words),
        mesh=mesh,
        out_type=jax.ShapeDtypeStruct((x_words,), jnp.float32),
        scratch_types=[
            pltpu.VMEM((_CHUNK,), jnp.float32),
            pltpu.VMEM((_CHUNK,), jnp.float32),
            pltpu.VMEM((_CHUNK,), jnp.float32),
            pltpu.VMEM((_CHUNK,), jnp.float32),
            pltpu.VMEM((_CHUNK,), jnp.float32),
            pltpu.VMEM((_CHUNK,), jnp.float32),
            pltpu.SemaphoreType.DMA,
            pltpu.SemaphoreType.DMA,
            pltpu.SemaphoreType.DMA,
            pltpu.SemaphoreType.DMA,
        ],
    )
    out = sc_call(x.reshape(-1), pos.reshape(-1))
    return out.reshape(B, T, C)


# speedup vs baseline: 4.9909x; 4.9909x over previous
import jax
import jax.numpy as jnp
from jax.experimental import pallas as pl


_ROWS = 512


def _add_kernel(x_ref, pos_ref, out_ref):
    out_ref[...] = x_ref[...] + pos_ref[...][None, :, :]


def kernel(x, position_embeddings):
    B, T, C = x.shape
    pos = position_embeddings[:T]
    grid = (T // _ROWS,)
    return pl.pallas_call(
        _add_kernel,
        grid=grid,
        in_specs=[
            pl.BlockSpec((B, _ROWS, C), lambda t: (0, t, 0)),
            pl.BlockSpec((_ROWS, C), lambda t: (t, 0)),
        ],
        out_specs=pl.BlockSpec((B, _ROWS, C), lambda t: (0, t, 0)),
        out_shape=jax.ShapeDtypeStruct((B, T, C), x.dtype),
    )(x, pos)
